# baseline (speedup 1.0000x reference)
import functools

import jax
import jax.numpy as jnp
from jax import lax
from jax.experimental import pallas as pl
from jax.experimental.pallas import tpu as pltpu
from jax.experimental.pallas import tpu_sc as plsc

_D = 128
_B = 16384
_NC = 2
_NS = 16
_NW = _NC * _NS
_BPW = _B // _NW
_NCH = 4
_CH = _BPW // _NCH

_mesh = plsc.VectorSubcoreMesh(core_axis_name="c", subcore_axis_name="s")


@functools.partial(
    pl.kernel,
    mesh=_mesh,
    out_type=jax.ShapeDtypeStruct((1, _B, _D), jnp.float32),
    scratch_types=[
        pltpu.VMEM_SHARED((_NS, 2, _D), jnp.float32),
        pltpu.VMEM((_BPW,), jnp.int32),
        pltpu.VMEM((_BPW, _D), jnp.float32),
    ] + [pltpu.SemaphoreType.DMA] * (2 * _NCH + 3),
)
def _gather_kernel(pres_hbm, abs_hbm, idx_hbm, out_hbm,
                   table_s, flags_v, rows_v, sem_p, sem_a, sem_o, *ksem):
    cid = lax.axis_index("c")
    sid = lax.axis_index("s")
    wid = sid * _NC + cid
    base = wid * _BPW
    out2d = out_hbm.at[0]
    sem_f = ksem[0]
    gsem = list(ksem[1:])
    cp_p = pltpu.async_copy(pres_hbm, table_s.at[sid].at[0], sem_p)
    cp_a = pltpu.async_copy(abs_hbm, table_s.at[sid].at[1], sem_a)
    cp_f = pltpu.async_copy(idx_hbm.at[pl.ds(base, _BPW)], flags_v, sem_f)
    cp_p.wait()
    cp_a.wait()
    cp_f.wait()
    gaths = []
    for k in range(_NCH):
        gaths.append(pltpu.async_copy(
            table_s.at[sid].at[flags_v.at[pl.ds(k * _CH, _CH)]],
            rows_v.at[pl.ds(k * _CH, _CH)], gsem[k]))
    outs = []
    for k in range(_NCH):
        gaths[k].wait()
        outs.append(pltpu.async_copy(
            rows_v.at[pl.ds(k * _CH, _CH)],
            out2d.at[pl.ds(base + k * _CH, _CH)], sem_o))
    for o in outs:
        o.wait()


def kernel(is_absent, present_goal_vector, absent_goal_vector):
    idx = is_absent.astype(jnp.int32)
    return _gather_kernel(present_goal_vector, absent_goal_vector, idx)
